# baseline (speedup 1.0000x reference)
import jax
import jax.numpy as jnp
from jax import lax
from jax.experimental import pallas as pl
from jax.experimental.pallas import tpu as pltpu
from jax.experimental.pallas import tpu_sc as plsc

NUM_ENTITIES = 1000000
NUM_RELATIONS = 1000
EMBED_DIM = 64
BATCH = 16384

NC, NS, L = 2, 16, 16
NW = NC * NS
D = EMBED_DIM
TOT = 2 * BATCH
UNIT = 512
NUNITS = -(-NUM_ENTITIES // UNIT)
LASTU = NUM_ENTITIES // UNIT
LASTW = NUM_ENTITIES - LASTU * UNIT
UPT = -(-NUNITS // NW)
GRPS = TOT // L
RPW = BATCH // NW


def _extract_body(sall_hbm, ent_hbm, out_hbm, sv, slab, stage, st_s,
                  sem_o, sem_s):
    wid = lax.axis_index("s") * NC + lax.axis_index("c")
    lo_e = wid * (UPT * UNIT)
    hi_e = jnp.minimum(lo_e + UPT * UNIT, NUM_ENTITIES)

    pltpu.sync_copy(sall_hbm, sv)
    st_s[0] = -1
    st_s[1] = 0
    iota = lax.iota(jnp.int32, L)

    def window(wi, carry):
        for q in range(4):
            kb = wi * 64 + q * L
            ev = sv[pl.ds(kb, L)]
            mi = jnp.where((ev >= lo_e) & (ev < hi_e), 1, 0)
            anyhit = jnp.sum(mi) > 0

            @pl.when(anyhit)
            def _():
                for j in range(L):
                    @pl.when(mi[j] > 0)
                    def _():
                        e = ev[j]
                        u = e // UNIT
                        p = e - u * UNIT

                        @pl.when(u != st_s[0])
                        def _():
                            st_s[0] = u
                            cs = pl.multiple_of(u * UNIT, 128)

                            @pl.when(u == LASTU)
                            def _():
                                pltpu.sync_copy(
                                    ent_hbm.at[:, pl.ds(cs, 128)],
                                    slab.at[:, pl.ds(0, 128)])

                            @pl.when(u != LASTU)
                            def _():
                                for a in range(8):
                                    pltpu.async_copy(
                                        ent_hbm.at[pl.ds(8 * a, 8),
                                                   pl.ds(cs, UNIT)],
                                        slab.at[pl.ds(8 * a, 8), :], sem_s)
                                for a in range(8):
                                    pltpu.make_async_copy(
                                        ent_hbm.at[pl.ds(0, 8),
                                                   pl.ds(0, UNIT)],
                                        slab.at[pl.ds(8 * a, 8), :],
                                        sem_s).wait()

                        n = st_s[1]
                        slot = lax.rem(n, 4)

                        @pl.when(n >= 4)
                        def _():
                            pltpu.make_async_copy(
                                out_hbm.at[pl.ds(0, D)],
                                stage.at[pl.ds(slot * D, D)], sem_o).wait()

                        colv = jnp.full((L,), p, jnp.int32)
                        for g in range(D // L):
                            v = plsc.load_gather(slab, [g * L + iota, colv])
                            stage[pl.ds(slot * D + g * L, L)] = v
                        pltpu.async_copy(
                            stage.at[pl.ds(slot * D, D)],
                            out_hbm.at[pl.ds((kb + j) * D, D)], sem_o)
                        st_s[1] = n + 1
        return carry

    lax.fori_loop(0, GRPS // 4, window, 0)

    nfin = jnp.minimum(st_s[1], 4)

    def _drain_cond(i):
        return i < nfin

    def _drain(i):
        pltpu.make_async_copy(out_hbm.at[pl.ds(0, D)],
                              stage.at[pl.ds(0, D)], sem_o).wait()
        return i + 1

    lax.while_loop(_drain_cond, _drain, 0)


def _score_body(ih_hbm, ir_hbm, it_hbm, cat_hbm, rel_hbm, out_hbm,
                ihv, irv, itv, h_v, r_v, t_v, score_v, sem):
    wid = lax.axis_index("s") * NC + lax.axis_index("c")
    base = wid * RPW

    pltpu.sync_copy(ih_hbm.at[pl.ds(base, RPW)], ihv)
    pltpu.sync_copy(ir_hbm.at[pl.ds(base, RPW)], irv)
    pltpu.sync_copy(it_hbm.at[pl.ds(base, RPW)], itv)

    ch = pltpu.async_copy(cat_hbm.at[ihv], h_v, sem)
    ct = pltpu.async_copy(cat_hbm.at[itv], t_v, sem)
    cr = pltpu.async_copy(rel_hbm.at[irv], r_v, sem)
    ch.wait()
    ct.wait()
    cr.wait()

    CH = D // L
    iota = lax.iota(jnp.int32, L)
    GROUPS = RPW // L

    def group(g, carry):
        score_vec = jnp.zeros((L,), jnp.float32)
        for b in range(L):
            row = g * L + b
            acc = None
            for c in range(CH):
                hv = h_v[row, pl.ds(c * L, L)]
                rv = r_v[row, pl.ds(c * L, L)]
                tv = t_v[row, pl.ds(c * L, L)]
                p = hv * rv * tv
                acc = p if acc is None else acc + p
            s = jnp.sum(acc)
            score_vec = jnp.where(iota == b, s, score_vec)
        score_v[pl.ds(g * L, L)] = score_vec
        return carry

    lax.fori_loop(0, GROUPS, group, 0)

    pltpu.sync_copy(score_v, out_hbm.at[pl.ds(base, RPW)])


@jax.jit
def kernel(h_indices, r_indices, t_indices, all_entity_embeddings,
           relation_embeds):
    mesh = plsc.VectorSubcoreMesh(core_axis_name="c", subcore_axis_name="s",
                                  num_cores=NC, num_subcores=NS)
    extract = pl.kernel(
        _extract_body,
        out_type=jax.ShapeDtypeStruct((TOT * D,), jnp.float32),
        mesh=mesh,
        compiler_params=pltpu.CompilerParams(needs_layout_passes=False),
        scratch_types=[
            pltpu.VMEM((TOT,), jnp.int32),
            pltpu.VMEM((D, UNIT), jnp.float32),
            pltpu.VMEM((4 * D,), jnp.float32),
            pltpu.SMEM((8,), jnp.int32),
            pltpu.SemaphoreType.DMA,
            pltpu.SemaphoreType.DMA,
        ],
    )
    score = pl.kernel(
        _score_body,
        out_type=jax.ShapeDtypeStruct((BATCH,), jnp.float32),
        mesh=mesh,
        compiler_params=pltpu.CompilerParams(needs_layout_passes=False,
                                             use_tc_tiling_on_sc=False),
        scratch_types=[
            pltpu.VMEM((RPW,), jnp.int32),
            pltpu.VMEM((RPW,), jnp.int32),
            pltpu.VMEM((RPW,), jnp.int32),
            pltpu.VMEM((RPW, D), jnp.float32),
            pltpu.VMEM((RPW, D), jnp.float32),
            pltpu.VMEM((RPW, D), jnp.float32),
            pltpu.VMEM((RPW,), jnp.float32),
            pltpu.SemaphoreType.DMA,
        ],
    )

    cat = jnp.concatenate([h_indices, t_indices])
    order = jnp.argsort(cat)
    sall = cat[order]
    inv = jnp.zeros((TOT,), jnp.int32).at[order].set(
        jnp.arange(TOT, dtype=jnp.int32))
    invh = inv[:BATCH]
    invt = inv[BATCH:]

    flat = extract(sall, all_entity_embeddings.T)
    cat2d = flat.reshape(TOT, D)
    return score(invh, r_indices, invt, cat2d, relation_embeds)


# speedup vs baseline: 1.4198x; 1.4198x over previous
import jax
import jax.numpy as jnp
from jax import lax
from jax.experimental import pallas as pl
from jax.experimental.pallas import tpu as pltpu
from jax.experimental.pallas import tpu_sc as plsc

NUM_ENTITIES = 1000000
NUM_RELATIONS = 1000
EMBED_DIM = 64
BATCH = 16384

NC, NS, L = 2, 16, 16
NW = NC * NS
D = EMBED_DIM
TOT = 2 * BATCH
UNIT = 1024
NUNITS = -(-NUM_ENTITIES // UNIT)
LASTU = NUM_ENTITIES // UNIT
LASTW = NUM_ENTITIES - LASTU * UNIT
UPT = -(-NUNITS // NW)
GRPS = TOT // L
RPW = BATCH // NW


def _extract_body(sall_hbm, ent_hbm, out_hbm, sv, slab, stage, st_s,
                  sem_o, sem_s):
    wid = lax.axis_index("s") * NC + lax.axis_index("c")
    lo_e = wid * (UPT * UNIT)
    hi_e = jnp.minimum(lo_e + UPT * UNIT, NUM_ENTITIES)

    pltpu.sync_copy(sall_hbm, sv)
    st_s[0] = -1
    st_s[1] = 0
    iota = lax.iota(jnp.int32, L)

    def window(wi, carry):
        for q in range(4):
            kb = wi * 64 + q * L
            ev = sv[pl.ds(kb, L)]
            mi = jnp.where((ev >= lo_e) & (ev < hi_e), 1, 0)
            anyhit = (ev[0] < hi_e) & (ev[L - 1] >= lo_e)

            @pl.when(anyhit)
            def _():
                for j in range(L):
                    @pl.when(mi[j] > 0)
                    def _():
                        e = ev[j]
                        u = e // UNIT
                        p = e - u * UNIT

                        @pl.when(u != st_s[0])
                        def _():
                            st_s[0] = u
                            cs = pl.multiple_of(u * UNIT, 128)

                            @pl.when(u == LASTU)
                            def _():
                                pltpu.sync_copy(
                                    ent_hbm.at[:, pl.ds(cs, 640)],
                                    slab.at[:, pl.ds(0, 640)])

                            @pl.when(u != LASTU)
                            def _():
                                pltpu.sync_copy(
                                    ent_hbm.at[:, pl.ds(cs, UNIT)], slab)

                        n = st_s[1]
                        slot = lax.rem(n, 4)

                        @pl.when(n >= 4)
                        def _():
                            pltpu.make_async_copy(
                                out_hbm.at[pl.ds(0, D)],
                                stage.at[pl.ds(slot * D, D)], sem_o).wait()

                        colv = jnp.full((L,), p, jnp.int32)
                        for g in range(D // L):
                            v = plsc.load_gather(slab, [g * L + iota, colv])
                            stage[pl.ds(slot * D + g * L, L)] = v
                        pltpu.async_copy(
                            stage.at[pl.ds(slot * D, D)],
                            out_hbm.at[pl.ds((kb + j) * D, D)], sem_o)
                        st_s[1] = n + 1
        return carry

    lax.fori_loop(0, GRPS // 4, window, 0)

    nfin = jnp.minimum(st_s[1], 4)

    def _drain_cond(i):
        return i < nfin

    def _drain(i):
        pltpu.make_async_copy(out_hbm.at[pl.ds(0, D)],
                              stage.at[pl.ds(0, D)], sem_o).wait()
        return i + 1

    lax.while_loop(_drain_cond, _drain, 0)


def _score_body(ih_hbm, ir_hbm, it_hbm, cat_hbm, rel_hbm, out_hbm,
                ihv, irv, itv, h_v, r_v, t_v, score_v, sem):
    wid = lax.axis_index("s") * NC + lax.axis_index("c")
    base = wid * RPW

    pltpu.sync_copy(ih_hbm.at[pl.ds(base, RPW)], ihv)
    pltpu.sync_copy(ir_hbm.at[pl.ds(base, RPW)], irv)
    pltpu.sync_copy(it_hbm.at[pl.ds(base, RPW)], itv)

    ch = pltpu.async_copy(cat_hbm.at[ihv], h_v, sem)
    ct = pltpu.async_copy(cat_hbm.at[itv], t_v, sem)
    cr = pltpu.async_copy(rel_hbm.at[irv], r_v, sem)
    ch.wait()
    ct.wait()
    cr.wait()

    CH = D // L
    iota = lax.iota(jnp.int32, L)
    GROUPS = RPW // L

    def group(g, carry):
        score_vec = jnp.zeros((L,), jnp.float32)
        for b in range(L):
            row = g * L + b
            acc = None
            for c in range(CH):
                hv = h_v[row, pl.ds(c * L, L)]
                rv = r_v[row, pl.ds(c * L, L)]
                tv = t_v[row, pl.ds(c * L, L)]
                p = hv * rv * tv
                acc = p if acc is None else acc + p
            s = jnp.sum(acc)
            score_vec = jnp.where(iota == b, s, score_vec)
        score_v[pl.ds(g * L, L)] = score_vec
        return carry

    lax.fori_loop(0, GROUPS, group, 0)

    pltpu.sync_copy(score_v, out_hbm.at[pl.ds(base, RPW)])


@jax.jit
def kernel(h_indices, r_indices, t_indices, all_entity_embeddings,
           relation_embeds):
    mesh = plsc.VectorSubcoreMesh(core_axis_name="c", subcore_axis_name="s",
                                  num_cores=NC, num_subcores=NS)
    extract = pl.kernel(
        _extract_body,
        out_type=jax.ShapeDtypeStruct((TOT * D,), jnp.float32),
        mesh=mesh,
        compiler_params=pltpu.CompilerParams(needs_layout_passes=False),
        scratch_types=[
            pltpu.VMEM((TOT,), jnp.int32),
            pltpu.VMEM((D, UNIT), jnp.float32),
            pltpu.VMEM((4 * D,), jnp.float32),
            pltpu.SMEM((8,), jnp.int32),
            pltpu.SemaphoreType.DMA,
            pltpu.SemaphoreType.DMA,
        ],
    )
    score = pl.kernel(
        _score_body,
        out_type=jax.ShapeDtypeStruct((BATCH,), jnp.float32),
        mesh=mesh,
        compiler_params=pltpu.CompilerParams(needs_layout_passes=False,
                                             use_tc_tiling_on_sc=False),
        scratch_types=[
            pltpu.VMEM((RPW,), jnp.int32),
            pltpu.VMEM((RPW,), jnp.int32),
            pltpu.VMEM((RPW,), jnp.int32),
            pltpu.VMEM((RPW, D), jnp.float32),
            pltpu.VMEM((RPW, D), jnp.float32),
            pltpu.VMEM((RPW, D), jnp.float32),
            pltpu.VMEM((RPW,), jnp.float32),
            pltpu.SemaphoreType.DMA,
        ],
    )

    cat = jnp.concatenate([h_indices, t_indices])
    order = jnp.argsort(cat)
    sall = cat[order]
    inv = jnp.zeros((TOT,), jnp.int32).at[order].set(
        jnp.arange(TOT, dtype=jnp.int32))
    invh = inv[:BATCH]
    invt = inv[BATCH:]

    flat = extract(sall, all_entity_embeddings.T)
    cat2d = flat.reshape(TOT, D)
    return score(invh, r_indices, invt, cat2d, relation_embeds)
